# baseline (speedup 1.0000x reference)
import functools

import jax
import jax.numpy as jnp
from jax.experimental import pallas as pl
from jax.experimental.pallas import tpu as pltpu


def _se_kernel(wd_ref, bd_ref, wut_ref, bu_ref, x_ref, o_ref):
    x = x_ref[...]

    pooled = jnp.mean(x, axis=0, dtype=jnp.float32)

    h = jax.lax.dot_general(pooled, wd_ref[...], (((1,), (1,)), ((), ())),
                            preferred_element_type=jnp.float32)
    h = jnp.maximum(h + bd_ref[...], 0.0)
    s = jnp.dot(h, wut_ref[...], preferred_element_type=jnp.float32)
    s = jax.nn.sigmoid(s + bu_ref[...])

    o_ref[...] = x * s[None, :, :].astype(x.dtype)


def _pick(total, candidates):
    for c in candidates:
        if total % c == 0:
            return c
    return 1


def kernel(x_nchw, w_down, b_down, w_up, b_up):
    B, C, H, W = x_nchw.shape
    HW = H * W
    I = w_down.shape[0]
    dtype = x_nchw.dtype

    x_t = jnp.transpose(x_nchw, (2, 3, 0, 1)).reshape(HW, B, C)

    wd = w_down.astype(jnp.float32)
    bd2 = b_down.astype(jnp.float32).reshape(1, I)
    wut = w_up.astype(jnp.float32).T
    bu2 = b_up.astype(jnp.float32).reshape(1, C)

    hwt = _pick(HW, (49,))
    vmem = 100 << 20

    out = pl.pallas_call(
        _se_kernel,
        out_shape=jax.ShapeDtypeStruct((HW, B, C), dtype),
        grid_spec=pltpu.PrefetchScalarGridSpec(
            num_scalar_prefetch=0,
            grid=(HW // hwt,),
            in_specs=[
                pl.BlockSpec((I, C), lambda b: (0, 0)),
                pl.BlockSpec((1, I), lambda b: (0, 0)),
                pl.BlockSpec((I, C), lambda b: (0, 0)),
                pl.BlockSpec((1, C), lambda b: (0, 0)),
                pl.BlockSpec((hwt, B, C), lambda b: (b, 0, 0)),
            ],
            out_specs=pl.BlockSpec((hwt, B, C), lambda b: (b, 0, 0)),
        ),
        compiler_params=pltpu.CompilerParams(
            dimension_semantics=("parallel",),
            vmem_limit_bytes=vmem,
        ),
    )(wd, bd2, wut, bu2, x_t)

    return out.reshape(H, W, B, C).transpose(2, 3, 0, 1)


# speedup vs baseline: 1.0253x; 1.0253x over previous
import functools

import jax
import jax.numpy as jnp
from jax.experimental import pallas as pl
from jax.experimental.pallas import tpu as pltpu


def _se_kernel(wd_ref, bd_ref, wut_ref, bu_ref, x_ref, o_ref):
    x = x_ref[...]

    pooled = jnp.mean(x, axis=0, dtype=jnp.float32)

    h = jax.lax.dot_general(pooled, wd_ref[...], (((1,), (1,)), ((), ())),
                            preferred_element_type=jnp.float32)
    h = jnp.maximum(h + bd_ref[...], 0.0)
    s = jnp.dot(h, wut_ref[...], preferred_element_type=jnp.float32)
    s = jax.nn.sigmoid(s + bu_ref[...])

    o_ref[...] = x * s[None, :, :].astype(x.dtype)


def _pick(total, candidates):
    for c in candidates:
        if total % c == 0:
            return c
    return 1


def kernel(x_nchw, w_down, b_down, w_up, b_up):
    B, C, H, W = x_nchw.shape
    HW = H * W
    I = w_down.shape[0]
    dtype = x_nchw.dtype

    x_t = jnp.transpose(x_nchw, (2, 3, 0, 1)).reshape(HW, B, C)

    wd = w_down.astype(jnp.float32)
    bd2 = b_down.astype(jnp.float32).reshape(1, I)
    wut = w_up.astype(jnp.float32).T
    bu2 = b_up.astype(jnp.float32).reshape(1, C)

    nb = _pick(B, (32, 16, 8, 4, 2))
    vmem = 100 << 20

    out = pl.pallas_call(
        _se_kernel,
        out_shape=jax.ShapeDtypeStruct((HW, B, C), dtype),
        grid_spec=pltpu.PrefetchScalarGridSpec(
            num_scalar_prefetch=0,
            grid=(B // nb,),
            in_specs=[
                pl.BlockSpec((I, C), lambda b: (0, 0)),
                pl.BlockSpec((1, I), lambda b: (0, 0)),
                pl.BlockSpec((I, C), lambda b: (0, 0)),
                pl.BlockSpec((1, C), lambda b: (0, 0)),
                pl.BlockSpec((HW, nb, C), lambda b: (0, b, 0)),
            ],
            out_specs=pl.BlockSpec((HW, nb, C), lambda b: (0, b, 0)),
        ),
        compiler_params=pltpu.CompilerParams(
            dimension_semantics=("parallel",),
            vmem_limit_bytes=vmem,
        ),
    )(wd, bd2, wut, bu2, x_t)

    return out.reshape(H, W, B, C).transpose(2, 3, 0, 1)


# speedup vs baseline: 1.0313x; 1.0058x over previous
import jax
import jax.numpy as jnp
from jax.experimental import pallas as pl
from jax.experimental.pallas import tpu as pltpu


def _se_kernel(wd_ref, bd_ref, wut_ref, bu_ref, x_ref, o_ref):
    x = x_ref[...]

    pooled = jnp.mean(x, axis=0, dtype=jnp.float32)

    h = jax.lax.dot_general(pooled, wd_ref[...], (((1,), (1,)), ((), ())),
                            preferred_element_type=jnp.float32)
    h = jnp.maximum(h + bd_ref[...], 0.0)
    s = jnp.dot(h, wut_ref[...], preferred_element_type=jnp.float32)
    s = jax.nn.sigmoid(s + bu_ref[...])

    o_ref[...] = x * s[None, :, :].astype(x.dtype)


def _pick(total, candidates):
    for c in candidates:
        if total % c == 0:
            return c
    return 1


def kernel(x_nchw, w_down, b_down, w_up, b_up):
    B, C, H, W = x_nchw.shape
    HW = H * W
    I = w_down.shape[0]
    dtype = x_nchw.dtype

    x_t = jnp.transpose(x_nchw, (2, 3, 0, 1)).reshape(HW, B, C)

    wd = w_down.astype(jnp.float32)
    bd2 = b_down.astype(jnp.float32).reshape(1, I)
    wut = w_up.astype(jnp.float32).T
    bu2 = b_up.astype(jnp.float32).reshape(1, C)

    nb = _pick(B, (32, 16, 8, 4, 2))
    vmem = 100 << 20

    out = pl.pallas_call(
        _se_kernel,
        out_shape=jax.ShapeDtypeStruct((HW, B, C), dtype),
        grid_spec=pltpu.PrefetchScalarGridSpec(
            num_scalar_prefetch=0,
            grid=(B // nb,),
            in_specs=[
                pl.BlockSpec((I, C), lambda b: (0, 0)),
                pl.BlockSpec((1, I), lambda b: (0, 0)),
                pl.BlockSpec((I, C), lambda b: (0, 0)),
                pl.BlockSpec((1, C), lambda b: (0, 0)),
                pl.BlockSpec((HW, nb, C), lambda b: (0, b, 0)),
            ],
            out_specs=pl.BlockSpec((HW, nb, C), lambda b: (0, b, 0)),
        ),
        compiler_params=pltpu.CompilerParams(
            dimension_semantics=("parallel",),
            vmem_limit_bytes=vmem,
        ),
    )(wd, bd2, wut, bu2, x_t)

    return out.reshape(H, W, B, C).transpose(2, 3, 0, 1)
